# baseline (speedup 1.0000x reference)
import functools

import jax
import jax.numpy as jnp
from jax import lax
from jax.experimental import pallas as pl
from jax.experimental.pallas import tpu as pltpu
from jax.experimental.pallas import tpu_sc as plsc

NUM_TYPES = 32
EMB_DIM = 16
N_EDGES = 3200000

_info = plsc.get_sparse_core_info()
_NC, _NS = _info.num_cores, _info.num_subcores
_NW = _NC * _NS
_NPART = 5
_CHUNK = 2000
_GROUPS = _CHUNK // 16


def _make_kernel(n_edges):
    per_w = n_edges // _NW
    n_iter = per_w // _CHUNK
    n_half = n_iter // 2
    mesh = plsc.VectorSubcoreMesh(core_axis_name="c", subcore_axis_name="s")

    @functools.partial(
        pl.kernel,
        mesh=mesh,
        compiler_params=pltpu.CompilerParams(
            use_tc_tiling_on_sc=False, needs_layout_passes=False),
        out_type=jax.ShapeDtypeStruct((EMB_DIM, n_edges), jnp.float32),
        scratch_types=[
            pltpu.VMEM((NUM_TYPES * EMB_DIM,), jnp.float32),
            pltpu.VMEM((_CHUNK,), jnp.int32),
            pltpu.VMEM((_CHUNK,), jnp.int32),
            pltpu.VMEM((EMB_DIM, _CHUNK), jnp.float32),
            pltpu.VMEM((EMB_DIM, _CHUNK), jnp.float32),
        ] + [pltpu.SemaphoreType.DMA] * 4,
    )
    def gather_kernel(table_hbm, idx_hbm, out_hbm,
                      table_v, idx0, idx1, rows0, rows1, si0, si1, so0, so1):
        wid = lax.axis_index("s") * _NC + lax.axis_index("c")
        w_base = wid * per_w
        idx_b, rows_b = (idx0, idx1), (rows0, rows1)
        si, so = (si0, si1), (so0, so1)

        def ibase(g):
            return w_base + g * _CHUNK

        pltpu.sync_copy(table_hbm, table_v)
        pltpu.async_copy(idx_hbm.at[pl.ds(ibase(0), _CHUNK)], idx0, si0)

        def compute_chunk(idx_ref, rows_ref):
            def jbody(j, carry):
                idxvec = idx_ref[pl.ds(j * 16, 16)]
                cols = [
                    plsc.load_gather(table_v, [idxvec + c * NUM_TYPES])
                    for c in range(EMB_DIM)
                ]
                for c in range(EMB_DIM):
                    rows_ref[c, pl.ds(j * 16, 16)] = cols[c]
                return carry
            lax.fori_loop(0, _GROUPS, jbody, 0)

        def step(g, b, not_first, not_last):
            pltpu.make_async_copy(
                idx_hbm.at[pl.ds(ibase(g), _CHUNK)], idx_b[b], si[b]).wait()

            def next_idx_load():
                pltpu.async_copy(
                    idx_hbm.at[pl.ds(ibase(g + 1), _CHUNK)],
                    idx_b[1 - b], si[1 - b])
            if not_last is None:
                next_idx_load()
            elif not_last is False:
                pass
            else:
                pl.when(not_last)(next_idx_load)

            def wait_prev_out():
                pltpu.make_async_copy(
                    rows_b[b],
                    out_hbm.at[:, pl.ds(ibase(g - 2), _CHUNK)],
                    so[b]).wait()
            if not_first is None:
                wait_prev_out()
            else:
                pl.when(not_first)(wait_prev_out)

            compute_chunk(idx_b[b], rows_b[b])

            pltpu.async_copy(
                rows_b[b], out_hbm.at[:, pl.ds(ibase(g), _CHUNK)], so[b])

        odd_tail = (n_iter % 2) == 1

        def body(i, carry):
            g = 2 * i
            step(g, 0, i >= 1, None)
            step(g + 1, 1, i >= 1,
                 None if odd_tail else (i < n_half - 1))
            return carry

        lax.fori_loop(0, n_half, body, 0)

        if odd_tail:
            step(n_iter - 1, 0, None, False)
            last0, last1 = n_iter - 1, n_iter - 2
        else:
            last0, last1 = n_iter - 2, n_iter - 1

        pltpu.make_async_copy(
            rows0, out_hbm.at[:, pl.ds(ibase(last0), _CHUNK)], so0).wait()
        pltpu.make_async_copy(
            rows1, out_hbm.at[:, pl.ds(ibase(last1), _CHUNK)], so1).wait()

    return gather_kernel


_gather_part = _make_kernel(N_EDGES // _NPART)


def kernel(edge_attr, table):
    tab_t = table.T.reshape(-1)
    q = N_EDGES // _NPART
    parts = [_gather_part(tab_t, edge_attr[i * q:(i + 1) * q])
             for i in range(_NPART)]
    return jnp.concatenate(parts, axis=1).T


# speedup vs baseline: 1.0054x; 1.0054x over previous
import functools

import jax
import jax.numpy as jnp
from jax import lax
from jax.experimental import pallas as pl
from jax.experimental.pallas import tpu as pltpu
from jax.experimental.pallas import tpu_sc as plsc

NUM_TYPES = 32
EMB_DIM = 16
N_EDGES = 3200000

_info = plsc.get_sparse_core_info()
_NC, _NS = _info.num_cores, _info.num_subcores
_NW = _NC * _NS
_NPART = 1
_CHUNK = 2000
_GROUPS = _CHUNK // 16


def _make_kernel(n_edges):
    per_w = n_edges // _NW
    n_iter = per_w // _CHUNK
    n_half = n_iter // 2
    mesh = plsc.VectorSubcoreMesh(core_axis_name="c", subcore_axis_name="s")

    @functools.partial(
        pl.kernel,
        mesh=mesh,
        compiler_params=pltpu.CompilerParams(
            use_tc_tiling_on_sc=False, needs_layout_passes=False),
        out_type=jax.ShapeDtypeStruct((EMB_DIM, n_edges), jnp.float32),
        scratch_types=[
            pltpu.VMEM((NUM_TYPES * EMB_DIM,), jnp.float32),
            pltpu.VMEM((_CHUNK,), jnp.int32),
            pltpu.VMEM((_CHUNK,), jnp.int32),
            pltpu.VMEM((EMB_DIM, _CHUNK), jnp.float32),
            pltpu.VMEM((EMB_DIM, _CHUNK), jnp.float32),
        ] + [pltpu.SemaphoreType.DMA] * 4,
    )
    def gather_kernel(table_hbm, idx_hbm, out_hbm,
                      table_v, idx0, idx1, rows0, rows1, si0, si1, so0, so1):
        wid = lax.axis_index("s") * _NC + lax.axis_index("c")
        w_base = wid * per_w
        idx_b, rows_b = (idx0, idx1), (rows0, rows1)
        si, so = (si0, si1), (so0, so1)

        def ibase(g):
            return w_base + g * _CHUNK

        pltpu.sync_copy(table_hbm, table_v)
        pltpu.async_copy(idx_hbm.at[pl.ds(ibase(0), _CHUNK)], idx0, si0)

        def compute_chunk(idx_ref, rows_ref):
            def jbody(j, carry):
                idxvec = idx_ref[pl.ds(j * 16, 16)]
                cols = [
                    plsc.load_gather(table_v, [idxvec + c * NUM_TYPES])
                    for c in range(EMB_DIM)
                ]
                for c in range(EMB_DIM):
                    rows_ref[c, pl.ds(j * 16, 16)] = cols[c]
                return carry
            lax.fori_loop(0, _GROUPS, jbody, 0)

        def step(g, b, not_first, not_last):
            pltpu.make_async_copy(
                idx_hbm.at[pl.ds(ibase(g), _CHUNK)], idx_b[b], si[b]).wait()

            def next_idx_load():
                pltpu.async_copy(
                    idx_hbm.at[pl.ds(ibase(g + 1), _CHUNK)],
                    idx_b[1 - b], si[1 - b])
            if not_last is None:
                next_idx_load()
            elif not_last is False:
                pass
            else:
                pl.when(not_last)(next_idx_load)

            def wait_prev_out():
                pltpu.make_async_copy(
                    rows_b[b],
                    out_hbm.at[:, pl.ds(ibase(g - 2), _CHUNK)],
                    so[b]).wait()
            if not_first is None:
                wait_prev_out()
            else:
                pl.when(not_first)(wait_prev_out)

            compute_chunk(idx_b[b], rows_b[b])

            pltpu.async_copy(
                rows_b[b], out_hbm.at[:, pl.ds(ibase(g), _CHUNK)], so[b])

        odd_tail = (n_iter % 2) == 1

        def body(i, carry):
            g = 2 * i
            step(g, 0, i >= 1, None)
            step(g + 1, 1, i >= 1,
                 None if odd_tail else (i < n_half - 1))
            return carry

        lax.fori_loop(0, n_half, body, 0)

        if odd_tail:
            step(n_iter - 1, 0, None, False)
            last0, last1 = n_iter - 1, n_iter - 2
        else:
            last0, last1 = n_iter - 2, n_iter - 1

        pltpu.make_async_copy(
            rows0, out_hbm.at[:, pl.ds(ibase(last0), _CHUNK)], so0).wait()
        pltpu.make_async_copy(
            rows1, out_hbm.at[:, pl.ds(ibase(last1), _CHUNK)], so1).wait()

    return gather_kernel


_gather_part = _make_kernel(N_EDGES // _NPART)


def kernel(edge_attr, table):
    tab_t = table.T.reshape(-1)
    return _gather_part(tab_t, edge_attr).T


# speedup vs baseline: 27.6004x; 27.4522x over previous
import functools

import jax
import jax.numpy as jnp
from jax import lax
from jax.experimental import pallas as pl
from jax.experimental.pallas import tpu as pltpu
from jax.experimental.pallas import tpu_sc as plsc

NUM_TYPES = 32
EMB_DIM = 16
N_EDGES = 3200000

_info = plsc.get_sparse_core_info()
_NC, _NS = _info.num_cores, _info.num_subcores
_NW = _NC * _NS
_CHUNK = 1024
_NCHUNK = N_EDGES // _CHUNK
_STEPS = (_NCHUNK + _NW - 1) // _NW
_NHALF = _STEPS // 2
_NBLK = _CHUNK // 128
_EBLK = N_EDGES // 128


def _make_kernel():
    mesh = plsc.VectorSubcoreMesh(core_axis_name="c", subcore_axis_name="s")

    @functools.partial(
        pl.kernel,
        mesh=mesh,
        compiler_params=pltpu.CompilerParams(
            use_tc_tiling_on_sc=False, needs_layout_passes=False),
        out_type=jax.ShapeDtypeStruct((2, _EBLK, 8, 128), jnp.float32),
        scratch_types=[
            pltpu.VMEM((NUM_TYPES * EMB_DIM,), jnp.float32),
            pltpu.VMEM((_CHUNK,), jnp.int32),
            pltpu.VMEM((_CHUNK,), jnp.int32),
            pltpu.VMEM((_NBLK, 8, 128), jnp.float32),
            pltpu.VMEM((_NBLK, 8, 128), jnp.float32),
            pltpu.VMEM((_NBLK, 8, 128), jnp.float32),
            pltpu.VMEM((_NBLK, 8, 128), jnp.float32),
        ] + [pltpu.SemaphoreType.DMA] * 4,
    )
    def gather_kernel(table_hbm, idx_hbm, out_hbm,
                      table_v, idx0, idx1, lo0, lo1, hi0, hi1,
                      si0, si1, so0, so1):
        wid = lax.axis_index("s") * _NC + lax.axis_index("c")
        n_w = (_NCHUNK - wid + _NW - 1) // _NW
        cap = n_w - 1
        idx_b = (idx0, idx1)
        lo_b, hi_b = (lo0, lo1), (hi0, hi1)
        si, so = (si0, si1), (so0, so1)

        def cid(g):
            return wid + _NW * jnp.minimum(g, cap)

        pltpu.sync_copy(table_hbm, table_v)
        pltpu.async_copy(idx_hbm.at[pl.ds(cid(0) * _CHUNK, _CHUNK)],
                         idx0, si0)

        def compute_chunk(idx_ref, lo_ref, hi_ref):
            def bbody(blk, carry):
                for e16 in range(8):
                    idxvec = idx_ref[pl.ds(blk * 128 + e16 * 16, 16)]
                    cols = [
                        plsc.load_gather(table_v, [idxvec + c * NUM_TYPES])
                        for c in range(EMB_DIM)
                    ]
                    for c in range(8):
                        lo_ref[blk, c, pl.ds(e16 * 16, 16)] = cols[c]
                    for c in range(8):
                        hi_ref[blk, c, pl.ds(e16 * 16, 16)] = cols[8 + c]
                return carry
            lax.fori_loop(0, _NBLK, bbody, 0)

        def out_copies(g, b, issue):
            c = cid(g)
            mk = pltpu.make_async_copy
            d_lo = mk(lo_b[b], out_hbm.at[0, pl.ds(c * _NBLK, _NBLK)], so[b])
            d_hi = mk(hi_b[b], out_hbm.at[1, pl.ds(c * _NBLK, _NBLK)], so[b])
            if issue:
                d_lo.start()
                d_hi.start()
            else:
                d_lo.wait()
                d_hi.wait()

        def step(g, b, not_first, not_last):
            pltpu.make_async_copy(
                idx_hbm.at[pl.ds(cid(g) * _CHUNK, _CHUNK)],
                idx_b[b], si[b]).wait()

            def next_idx_load():
                pltpu.async_copy(
                    idx_hbm.at[pl.ds(cid(g + 1) * _CHUNK, _CHUNK)],
                    idx_b[1 - b], si[1 - b])
            if not_last is None:
                next_idx_load()
            elif not_last is False:
                pass
            else:
                pl.when(not_last)(next_idx_load)

            def wait_prev_out():
                out_copies(g - 2, b, issue=False)
            if not_first is None:
                wait_prev_out()
            else:
                pl.when(not_first)(wait_prev_out)

            compute_chunk(idx_b[b], lo_b[b], hi_b[b])
            out_copies(g, b, issue=True)

        def body(i, carry):
            g = 2 * i
            step(g, 0, i >= 1, None)
            step(g + 1, 1, i >= 1, i < _NHALF - 1)
            return carry

        lax.fori_loop(0, _NHALF, body, 0)

        out_copies(_STEPS - 2, 0, issue=False)
        out_copies(_STEPS - 1, 1, issue=False)

    return gather_kernel


_gather = _make_kernel()


def kernel(edge_attr, table):
    tab_t = table.T.reshape(-1)
    out4d = _gather(tab_t, edge_attr)
    return out4d.transpose(1, 3, 0, 2).reshape(N_EDGES, EMB_DIM)
